# baseline (speedup 1.0000x reference)
import functools

import numpy as np
import jax
import jax.numpy as jnp
from jax import lax
from jax.experimental import pallas as pl
from jax.experimental.pallas import tpu as pltpu
from jax.experimental.pallas import tpu_sc as plsc

N_VOCAB = 1000000
D = 64
SEQ = 200
BATCH = 1024
NTOK = BATCH * SEQ

NC = 2
NS = 16
NW = NC * NS
TOK_PER_W = NTOK // NW
CHUNK = 128
NCHUNK = TOK_PER_W // CHUNK
NGROUP = CHUNK // 16

_INV_D = 1.0 / D
_EPS = 1e-6
_SCALE = float(D) ** 0.5


def _pos_table_t() -> np.ndarray:
    pos = np.arange(SEQ, dtype=np.float64)[:, None]
    j = np.arange(D, dtype=np.float64)[None, :]
    angle = pos / np.power(10000.0, 2.0 * (np.floor(j / 2.0)) / D)
    table = angle.copy()
    table[:, 0::2] = np.sin(angle[:, 0::2])
    table[:, 1::2] = np.cos(angle[:, 1::2])
    t = table.astype(np.float32).T
    return np.concatenate([t, t[:, :16]], axis=1)


_POS_T = _pos_table_t()


def _rsqrt16(x):
    i = plsc.bitcast(x, jnp.int32)
    i = jnp.int32(0x5F3759DF) - lax.shift_right_logical(i, 1)
    y = plsc.bitcast(i, jnp.float32)
    for _ in range(3):
        y = y * (1.5 - 0.5 * x * y * y)
    return y


def _sc_body(src_hbm, emb_hbm, post_hbm, w_hbm, b_hbm, out_hbm,
             idx_v, rows_v, ebuf, post_v, w_v, b_v, sem):
    wid = lax.axis_index("s") * NC + lax.axis_index("c")
    tok0 = wid * TOK_PER_W

    pltpu.sync_copy(post_hbm, post_v)
    pltpu.sync_copy(w_hbm, w_v)
    pltpu.sync_copy(b_hbm, b_v)
    pltpu.sync_copy(src_hbm.at[pl.ds(tok0, TOK_PER_W)], idx_v)

    iota = lax.iota(jnp.int32, 16)
    wq = [w_v[pl.ds(q * 16, 16)] for q in range(D // 16)]
    bq = [b_v[pl.ds(q * 16, 16)] for q in range(D // 16)]

    def chunk_body(c, _):
        base = c * CHUNK
        pltpu.async_copy(
            emb_hbm.at[idx_v.at[pl.ds(base, CHUNK)]], rows_v, sem
        ).wait()

        def group_body(g, _):
            rowvec = g * 16 + iota
            l0 = lax.rem(base + g * 16, SEQ)
            acc_s = [None] * 4
            acc_q = [None] * 4
            for j in range(D):
                k = j % 4
                v = plsc.load_gather(rows_v, [rowvec, jnp.full((16,), j, jnp.int32)])
                p = post_v[j, pl.ds(l0, 16)]
                e = v * _SCALE + p
                ebuf[j, :] = e
                acc_s[k] = e if acc_s[k] is None else acc_s[k] + e
                acc_q[k] = e * e if acc_q[k] is None else acc_q[k] + e * e
            mean = ((acc_s[0] + acc_s[1]) + (acc_s[2] + acc_s[3])) * _INV_D
            q = (acc_q[0] + acc_q[1]) + (acc_q[2] + acc_q[3])
            var = q * _INV_D - mean * mean
            inv = _rsqrt16(var + _EPS)
            for j in range(D):
                e = ebuf[j, :]
                o = (e - mean) * inv * wq[j // 16][j % 16] + bq[j // 16][j % 16]
                plsc.store_scatter(rows_v, [rowvec, jnp.full((16,), j, jnp.int32)], o)
            return 0

        lax.fori_loop(0, NGROUP, group_body, 0)
        pltpu.sync_copy(rows_v, out_hbm.at[pl.ds(tok0 + base, CHUNK)])
        return 0

    lax.fori_loop(0, NCHUNK, chunk_body, 0)


@jax.jit
def _run(src_flat, emb_table, pos_t, ln_weight, ln_bias):
    mesh = plsc.VectorSubcoreMesh(core_axis_name="c", subcore_axis_name="s")
    k = functools.partial(
        pl.kernel,
        mesh=mesh,
        out_type=jax.ShapeDtypeStruct((NTOK, D), jnp.float32),
        compiler_params=pltpu.CompilerParams(
            needs_layout_passes=False, use_tc_tiling_on_sc=False
        ),
        scratch_types=[
            pltpu.VMEM((TOK_PER_W,), jnp.int32),
            pltpu.VMEM((CHUNK, D), jnp.float32),
            pltpu.VMEM((D, 16), jnp.float32),
            pltpu.VMEM(_POS_T.shape, jnp.float32),
            pltpu.VMEM((D,), jnp.float32),
            pltpu.VMEM((D,), jnp.float32),
            pltpu.SemaphoreType.DMA,
        ],
    )(_sc_body)
    return k(src_flat, emb_table, pos_t, ln_weight, ln_bias)


def kernel(src_seq, emb_table, ln_weight, ln_bias):
    src_flat = src_seq.reshape(-1).astype(jnp.int32)
    pos_t = jnp.asarray(_POS_T)
    out = _run(src_flat, emb_table, pos_t, ln_weight, ln_bias)
    return out.reshape(BATCH, SEQ, D)


# speedup vs baseline: 1.1766x; 1.1766x over previous
import functools

import numpy as np
import jax
import jax.numpy as jnp
from jax import lax
from jax.experimental import pallas as pl
from jax.experimental.pallas import tpu as pltpu
from jax.experimental.pallas import tpu_sc as plsc

N_VOCAB = 1000000
D = 64
SEQ = 200
BATCH = 1024
NTOK = BATCH * SEQ

NC = 2
NS = 16
NW = NC * NS
TOK_PER_W = NTOK // NW
CHUNK = 128
NCHUNK = TOK_PER_W // CHUNK
NGROUP = CHUNK // 16

_INV_D = 1.0 / D
_SCALE = float(D) ** 0.5
_EPS = 1e-6 / (D)


def _pos_np() -> np.ndarray:
    pos = np.arange(SEQ, dtype=np.float64)[:, None]
    j = np.arange(D, dtype=np.float64)[None, :]
    angle = pos / np.power(10000.0, 2.0 * (np.floor(j / 2.0)) / D)
    table = angle.copy()
    table[:, 0::2] = np.sin(angle[:, 0::2])
    table[:, 1::2] = np.cos(angle[:, 1::2])
    return (table / _SCALE).astype(np.float32)


_P = _pos_np()
_POS_T = np.concatenate([_P.T, _P.T[:, :16]], axis=1)


def _rsqrt16(x):
    i = plsc.bitcast(x, jnp.int32)
    i = jnp.int32(0x5F3759DF) - lax.shift_right_logical(i, 1)
    y = plsc.bitcast(i, jnp.float32)
    for _ in range(3):
        y = y * (1.5 - 0.5 * x * y * y)
    return y


def _sc_body(src_hbm, emb_hbm, post_hbm, w_hbm, b_hbm, out_hbm,
             idx_v, rows_v, ebuf_v, obuf_v, post_v, w_v, b_v, sem):
    wid = lax.axis_index("s") * NC + lax.axis_index("c")
    tok0 = wid * TOK_PER_W

    pltpu.sync_copy(post_hbm, post_v)
    pltpu.sync_copy(w_hbm, w_v)
    pltpu.sync_copy(b_hbm, b_v)
    pltpu.sync_copy(src_hbm.at[pl.ds(tok0, TOK_PER_W)], idx_v)

    iota = lax.iota(jnp.int32, 16)
    wv = [w_v[pl.ds(t * 16, 16)] for t in range(4)]
    bv = [b_v[pl.ds(t * 16, 16)] for t in range(4)]

    def chunk_body(c, _):
        base = c * CHUNK
        pltpu.async_copy(
            emb_hbm.at[idx_v.at[pl.ds(base, CHUNK)]], rows_v, sem
        ).wait()

        def group_body(g):
            g16 = g * 16
            rowvec = g16 + iota
            l0 = lax.rem(base + g16, SEQ)
            a_s = [None] * 4
            a_q = [None] * 4
            for j in range(D):
                k = j % 4
                colj = jnp.full((16,), j, jnp.int32)
                v = plsc.load_gather(rows_v, [rowvec, colj])
                p = post_v[j, pl.ds(l0, 16)]
                e = v + p
                plsc.store_scatter(ebuf_v, [rowvec, colj], e)
                a_s[k] = e if a_s[k] is None else a_s[k] + e
                a_q[k] = e * e if a_q[k] is None else a_q[k] + e * e
            mean = ((a_s[0] + a_s[1]) + (a_s[2] + a_s[3])) * _INV_D
            q = (a_q[0] + a_q[1]) + (a_q[2] + a_q[3])
            var = q * _INV_D - mean * mean
            inv = _rsqrt16(var + _EPS)
            for r in range(16):
                m = mean[r]
                s_ = inv[r]
                row = g16 + r
                for t in range(4):
                    e = ebuf_v[row, pl.ds(t * 16, 16)]
                    o = (e - m) * s_ * wv[t] + bv[t]
                    obuf_v[row, pl.ds(t * 16, 16)] = o

        plsc.parallel_loop(0, NGROUP, step=1, unroll=8)(group_body)
        pltpu.sync_copy(obuf_v, out_hbm.at[pl.ds(tok0 + base, CHUNK)])
        return 0

    lax.fori_loop(0, NCHUNK, chunk_body, 0)


@jax.jit
def _run(src_flat, emb_table, pos_t, ln_weight, ln_bias):
    mesh = plsc.VectorSubcoreMesh(core_axis_name="c", subcore_axis_name="s")
    k = functools.partial(
        pl.kernel,
        mesh=mesh,
        out_type=jax.ShapeDtypeStruct((NTOK, D), jnp.float32),
        compiler_params=pltpu.CompilerParams(
            needs_layout_passes=False, use_tc_tiling_on_sc=False
        ),
        scratch_types=[
            pltpu.VMEM((TOK_PER_W,), jnp.int32),
            pltpu.VMEM((CHUNK, D), jnp.float32),
            pltpu.VMEM((CHUNK, D), jnp.float32),
            pltpu.VMEM((CHUNK, D), jnp.float32),
            pltpu.VMEM(_POS_T.shape, jnp.float32),
            pltpu.VMEM((D,), jnp.float32),
            pltpu.VMEM((D,), jnp.float32),
            pltpu.SemaphoreType.DMA,
        ],
    )(_sc_body)
    return k(src_flat, emb_table, pos_t, ln_weight, ln_bias)


def kernel(src_seq, emb_table, ln_weight, ln_bias):
    src_flat = src_seq.reshape(-1).astype(jnp.int32)
    out = _run(src_flat, emb_table, jnp.asarray(_POS_T), ln_weight, ln_bias)
    return out.reshape(BATCH, SEQ, D)


# speedup vs baseline: 1.6833x; 1.4306x over previous
import functools

import numpy as np
import jax
import jax.numpy as jnp
from jax import lax
from jax.experimental import pallas as pl
from jax.experimental.pallas import tpu as pltpu
from jax.experimental.pallas import tpu_sc as plsc

N_VOCAB = 1000000
D = 64
SEQ = 200
BATCH = 1024
NTOK = BATCH * SEQ

NC = 2
NS = 16
NW = NC * NS
TOK_PER_W = NTOK // NW
CHUNK = 128
NCHUNK = TOK_PER_W // CHUNK

_EPS = 1e-6
_SCALE = float(D) ** 0.5

TC_ROWS = 1600
TC_GRID = NTOK // TC_ROWS


def _pos_np() -> np.ndarray:
    pos = np.arange(SEQ, dtype=np.float64)[:, None]
    j = np.arange(D, dtype=np.float64)[None, :]
    angle = pos / np.power(10000.0, 2.0 * (np.floor(j / 2.0)) / D)
    table = angle.copy()
    table[:, 0::2] = np.sin(angle[:, 0::2])
    table[:, 1::2] = np.cos(angle[:, 1::2])
    return table.astype(np.float32)


_POS_TILED = np.tile(_pos_np(), (TC_ROWS // SEQ, 1))


def _sc_body(src_hbm, emb_hbm, out_hbm, idx_v, rows0, rows1, gs0, gs1):
    wid = lax.axis_index("s") * NC + lax.axis_index("c")
    tok0 = wid * TOK_PER_W

    pltpu.sync_copy(src_hbm.at[pl.ds(tok0, TOK_PER_W)], idx_v)

    rows = (rows0, rows1)
    gs = (gs0, gs1)

    def gather(c, b):
        return pltpu.make_async_copy(
            emb_hbm.at[idx_v.at[pl.ds(c * CHUNK, CHUNK)]], rows[b], gs[b]
        )

    gather(0, 0).start()

    def chunk_body(c2, _):
        for b in (0, 1):
            c = c2 * 2 + b
            nb = 1 - b

            @pl.when(c + 1 < NCHUNK)
            def _():
                gather(c + 1, nb).start()

            gather(c, b).wait()
            pltpu.sync_copy(rows[b], out_hbm.at[pl.ds(tok0 + c * CHUNK, CHUNK)])
        return 0

    lax.fori_loop(0, NCHUNK // 2, chunk_body, 0)


@jax.jit
def _sc_gather(src_flat, emb_table):
    mesh = plsc.VectorSubcoreMesh(core_axis_name="c", subcore_axis_name="s")
    k = functools.partial(
        pl.kernel,
        mesh=mesh,
        out_type=jax.ShapeDtypeStruct((NTOK, D), jnp.float32),
        compiler_params=pltpu.CompilerParams(
            needs_layout_passes=False, use_tc_tiling_on_sc=False
        ),
        scratch_types=[
            pltpu.VMEM((TOK_PER_W,), jnp.int32),
            pltpu.VMEM((CHUNK, D), jnp.float32),
            pltpu.VMEM((CHUNK, D), jnp.float32),
            pltpu.SemaphoreType.DMA,
            pltpu.SemaphoreType.DMA,
        ],
    )(_sc_body)
    return k(src_flat, emb_table)


def _tc_body(x_ref, pos_ref, w_ref, b_ref, o_ref):
    e = x_ref[...] * _SCALE + pos_ref[...]
    mean = jnp.mean(e, axis=1, keepdims=True)
    var = jnp.mean(e * e, axis=1, keepdims=True) - mean * mean
    o = (e - mean) * lax.rsqrt(var + _EPS) * w_ref[...] + b_ref[...]
    o_ref[...] = o


@jax.jit
def _tc_ln(x, pos_tiled, w, b):
    return pl.pallas_call(
        _tc_body,
        out_shape=jax.ShapeDtypeStruct((NTOK, D), jnp.float32),
        grid=(TC_GRID,),
        in_specs=[
            pl.BlockSpec((TC_ROWS, D), lambda i: (i, 0)),
            pl.BlockSpec((TC_ROWS, D), lambda i: (0, 0)),
            pl.BlockSpec((1, D), lambda i: (0, 0)),
            pl.BlockSpec((1, D), lambda i: (0, 0)),
        ],
        out_specs=pl.BlockSpec((TC_ROWS, D), lambda i: (i, 0)),
    )(x, pos_tiled, w, b)


def kernel(src_seq, emb_table, ln_weight, ln_bias):
    src_flat = src_seq.reshape(-1).astype(jnp.int32)
    gathered = _sc_gather(src_flat, emb_table)
    out = _tc_ln(
        gathered,
        jnp.asarray(_POS_TILED),
        ln_weight.reshape(1, D),
        ln_bias.reshape(1, D),
    )
    return out.reshape(BATCH, SEQ, D)
